# baseline (speedup 1.0000x reference)
import functools

import jax
import jax.numpy as jnp
from jax import lax
from jax.experimental import pallas as pl
from jax.experimental.pallas import tpu as pltpu
from jax.experimental.pallas import tpu_sc as plsc

_NC = 2
_NS = 16
_NW = _NC * _NS


def _gather_body(per_w, idx_hbm, table_hbm, out_hbm, idx_v, rows_v, sem):
    wid = lax.axis_index("s") * _NC + lax.axis_index("c")
    base = wid * per_w
    pltpu.sync_copy(idx_hbm.at[pl.ds(base, per_w)], idx_v)

    @pl.loop(0, per_w, step=16)
    def _fire(g):
        v = idx_v[pl.ds(g, 16)]
        for j in range(16):
            pltpu.make_async_copy(
                table_hbm.at[pl.ds(v[j], 1)], rows_v.at[pl.ds(g + j, 1)], sem
            ).start()

    pltpu.make_async_copy(out_hbm.at[pl.ds(base, per_w)], rows_v, sem).wait()
    pltpu.sync_copy(rows_v, out_hbm.at[pl.ds(base, per_w)])


def _sc_gather(idx, table):
    m = idx.shape[0]
    d = table.shape[1]
    per_w = m // _NW
    mesh = plsc.VectorSubcoreMesh(core_axis_name="c", subcore_axis_name="s")
    kern = functools.partial(
        pl.kernel,
        mesh=mesh,
        out_type=jax.ShapeDtypeStruct((m, d), table.dtype),
        scratch_types=[
            pltpu.VMEM((per_w,), jnp.int32),
            pltpu.VMEM((per_w, d), table.dtype),
            pltpu.SemaphoreType.DMA,
        ],
    )(functools.partial(_gather_body, per_w))
    return kern(idx, table)


def _mlp_body(e_ref, w1_ref, b1_ref, w2_ref, b2_ref, o_ref):
    h = jnp.dot(e_ref[...], w1_ref[...], preferred_element_type=jnp.float32)
    h = jax.nn.gelu(h + b1_ref[...])
    o_ref[...] = jnp.dot(h, w2_ref[...], preferred_element_type=jnp.float32) + b2_ref[...]


def _tc_mlp(embeds, W1, b1, W2, b2, block_m=1024):
    m, k = embeds.shape
    h = W1.shape[1]
    n = W2.shape[1]
    grid = (m // block_m,)
    return pl.pallas_call(
        _mlp_body,
        grid=grid,
        in_specs=[
            pl.BlockSpec((block_m, k), lambda i: (i, 0)),
            pl.BlockSpec((k, h), lambda i: (0, 0)),
            pl.BlockSpec((1, h), lambda i: (0, 0)),
            pl.BlockSpec((h, n), lambda i: (0, 0)),
            pl.BlockSpec((1, n), lambda i: (0, 0)),
        ],
        out_specs=pl.BlockSpec((block_m, n), lambda i: (i, 0)),
        out_shape=jax.ShapeDtypeStruct((m, n), jnp.float32),
    )(embeds, W1, b1, W2, b2)


def kernel(token_ids, table, W1, b1, W2, b2):
    b, s = token_ids.shape
    n_out = W2.shape[1]
    idx = token_ids.T.reshape(-1)
    embeds = _sc_gather(idx, table)
    out = _tc_mlp(embeds, W1, b1.reshape(1, -1), W2, b2.reshape(1, -1))
    return out.reshape(s, b, n_out).transpose(1, 0, 2)
